# baseline (speedup 1.0000x reference)
import jax
import jax.numpy as jnp
from jax import lax
from jax.experimental import pallas as pl
from jax.experimental.pallas import tpu as pltpu
from jax.experimental.pallas import tpu_sc as plsc

N = 100000
D = 128
CH = 96
NCH = N // CH
TAIL = N - NCH * CH
NW = 32
TRIPS = -(-NCH // NW)
NPW = TRIPS * CH
NPW_LAST = N - (NW - 1) * NPW


def _fire_gathers(We, Wa, Wc, Wh, idx_e, idx_a, idx_c, idx_h, lo, rows, sem):
    pltpu.make_async_copy(We.at[idx_e.at[pl.ds(lo, CH)]],
                          rows.at[:, pl.ds(0, D)], sem).start()
    pltpu.make_async_copy(Wa.at[idx_a.at[pl.ds(lo, CH)]],
                          rows.at[:, pl.ds(D, D)], sem).start()
    pltpu.make_async_copy(Wc.at[idx_c.at[pl.ds(lo, CH)]],
                          rows.at[:, pl.ds(2 * D, D)], sem).start()
    pltpu.make_async_copy(Wh.at[idx_h.at[pl.ds(lo, CH)]],
                          rows.at[:, pl.ds(3 * D, D)], sem).start()


def _wait_gathers(We, Wa, Wc, Wh, idx_e, idx_a, idx_c, idx_h, lo, rows, sem):
    pltpu.make_async_copy(We.at[idx_e.at[pl.ds(lo, CH)]],
                          rows.at[:, pl.ds(0, D)], sem).wait()
    pltpu.make_async_copy(Wa.at[idx_a.at[pl.ds(lo, CH)]],
                          rows.at[:, pl.ds(D, D)], sem).wait()
    pltpu.make_async_copy(Wc.at[idx_c.at[pl.ds(lo, CH)]],
                          rows.at[:, pl.ds(2 * D, D)], sem).wait()
    pltpu.make_async_copy(Wh.at[idx_h.at[pl.ds(lo, CH)]],
                          rows.at[:, pl.ds(3 * D, D)], sem).wait()


def _emb_body(elem, arom, chg, hct, We, Wa, Wc, Wh, out,
              idx_e, idx_a, idx_c, idx_h, rows_a, rows_b,
              gsa, gsb, wsa, wsb):
    w = lax.axis_index("s") * 2 + lax.axis_index("c")
    nodebase = w * NPW

    @pl.when(w < NW - 1)
    def _():
        pltpu.sync_copy(elem.at[pl.ds(nodebase, NPW)], idx_e)
        pltpu.sync_copy(arom.at[pl.ds(nodebase, NPW)], idx_a)
        pltpu.sync_copy(chg.at[pl.ds(nodebase, NPW)], idx_c)
        pltpu.sync_copy(hct.at[pl.ds(nodebase, NPW)], idx_h)

    @pl.when(w == NW - 1)
    def _():
        pltpu.sync_copy(elem.at[pl.ds(nodebase, NPW_LAST)],
                        idx_e.at[pl.ds(0, NPW_LAST)])
        pltpu.sync_copy(arom.at[pl.ds(nodebase, NPW_LAST)],
                        idx_a.at[pl.ds(0, NPW_LAST)])
        pltpu.sync_copy(chg.at[pl.ds(nodebase, NPW_LAST)],
                        idx_c.at[pl.ds(0, NPW_LAST)])
        pltpu.sync_copy(hct.at[pl.ds(nodebase, NPW_LAST)],
                        idx_h.at[pl.ds(0, NPW_LAST)])

    def one_buf(t, rows, gsem, wsem):
        c = w * TRIPS + t
        fire = (t < TRIPS) & (c < NCH)

        @pl.when((t >= 2) & (c - 2 < NCH))
        def _():
            pltpu.make_async_copy(out.at[pl.ds(0, CH)], rows, wsem).wait()

        @pl.when(fire)
        def _():
            _fire_gathers(We, Wa, Wc, Wh, idx_e, idx_a, idx_c, idx_h,
                          t * CH, rows, gsem)

        return fire

    def finish_buf(t, rows, gsem, wsem, fire):
        c = w * TRIPS + t

        @pl.when(fire)
        def _():
            _wait_gathers(We, Wa, Wc, Wh, idx_e, idx_a, idx_c, idx_h,
                          t * CH, rows, gsem)
            pltpu.make_async_copy(rows, out.at[pl.ds(c * CH, CH)], wsem).start()

    def body2(j, carry):
        t0 = 2 * j
        t1 = t0 + 1
        f0 = one_buf(t0, rows_a, gsa, wsa)
        f1 = one_buf(t1, rows_b, gsb, wsb)
        finish_buf(t0, rows_a, gsa, wsa, f0)
        finish_buf(t1, rows_b, gsb, wsb, f1)
        return carry

    lax.fori_loop(0, (TRIPS + 1) // 2, body2, None)

    @pl.when(w * TRIPS + (TRIPS - 1) < NCH)
    def _():
        pltpu.make_async_copy(out.at[pl.ds(0, CH)], rows_a, wsa).wait()

    @pl.when(w == NW - 1)
    def _():
        lo = NPW_LAST - TAIL
        cps = [
            pltpu.make_async_copy(We.at[idx_e.at[pl.ds(lo, TAIL)]],
                                  rows_a.at[pl.ds(0, TAIL), pl.ds(0, D)], gsa),
            pltpu.make_async_copy(Wa.at[idx_a.at[pl.ds(lo, TAIL)]],
                                  rows_a.at[pl.ds(0, TAIL), pl.ds(D, D)], gsa),
            pltpu.make_async_copy(Wc.at[idx_c.at[pl.ds(lo, TAIL)]],
                                  rows_a.at[pl.ds(0, TAIL), pl.ds(2 * D, D)], gsa),
            pltpu.make_async_copy(Wh.at[idx_h.at[pl.ds(lo, TAIL)]],
                                  rows_a.at[pl.ds(0, TAIL), pl.ds(3 * D, D)], gsa),
        ]
        for cp in cps:
            cp.start()
        for cp in cps:
            cp.wait()
        pltpu.sync_copy(rows_a.at[pl.ds(0, TAIL)], out.at[pl.ds(NCH * CH, TAIL)])


def kernel(element, aromatic, charge, hcount,
           W_element, W_aromatic, W_charge, W_hcount):
    mesh = plsc.VectorSubcoreMesh(core_axis_name="c", subcore_axis_name="s")
    f = pl.kernel(
        _emb_body,
        mesh=mesh,
        out_type=jax.ShapeDtypeStruct((N, 4 * D), jnp.float32),
        scratch_types=[
            pltpu.VMEM((NPW,), jnp.int32),
            pltpu.VMEM((NPW,), jnp.int32),
            pltpu.VMEM((NPW,), jnp.int32),
            pltpu.VMEM((NPW,), jnp.int32),
            pltpu.VMEM((CH, 4 * D), jnp.float32),
            pltpu.VMEM((CH, 4 * D), jnp.float32),
            pltpu.SemaphoreType.DMA,
            pltpu.SemaphoreType.DMA,
            pltpu.SemaphoreType.DMA,
            pltpu.SemaphoreType.DMA,
        ],
    )
    return f(element, aromatic, charge, hcount,
             W_element, W_aromatic, W_charge, W_hcount)


# speedup vs baseline: 1.2942x; 1.2942x over previous
import jax
import jax.numpy as jnp
from jax import lax
from jax.experimental import pallas as pl
from jax.experimental.pallas import tpu as pltpu
from jax.experimental.pallas import tpu_sc as plsc

N = 100000
D = 128
CH = 120
NCH = N // CH
TAIL = N - NCH * CH
NW = 32
TRIPS = -(-NCH // NW)


def _idx_copies(elem, arom, chg, hct, c, bufs, sem):
    ie, ia, ic, ih = bufs
    return [
        pltpu.make_async_copy(elem.at[pl.ds(c * CH, CH)], ie, sem),
        pltpu.make_async_copy(arom.at[pl.ds(c * CH, CH)], ia, sem),
        pltpu.make_async_copy(chg.at[pl.ds(c * CH, CH)], ic, sem),
        pltpu.make_async_copy(hct.at[pl.ds(c * CH, CH)], ih, sem),
    ]


def _gathers(We, Wa, Wc, Wh, bufs, rows, sem):
    ie, ia, ic, ih = bufs
    return [
        pltpu.make_async_copy(We.at[ie], rows.at[:, pl.ds(0, D)], sem),
        pltpu.make_async_copy(Wa.at[ia], rows.at[:, pl.ds(D, D)], sem),
        pltpu.make_async_copy(Wc.at[ic], rows.at[:, pl.ds(2 * D, D)], sem),
        pltpu.make_async_copy(Wh.at[ih], rows.at[:, pl.ds(3 * D, D)], sem),
    ]


def _emb_body(elem, arom, chg, hct, We, Wa, Wc, Wh, out,
              iea, iaa, ica, iha, ieb, iab, icb, ihb, rows_a, rows_b,
              isa, isb, gsa, gsb, wsa, wsb):
    w = lax.axis_index("s") * 2 + lax.axis_index("c")
    bufs_a = (iea, iaa, ica, iha)
    bufs_b = (ieb, iab, icb, ihb)

    def guard(t):
        return (t < TRIPS) & (t * NW + w < NCH)

    @pl.when(guard(0))
    def _():
        for cp in _idx_copies(elem, arom, chg, hct, 0 * NW + w, bufs_a, isa):
            cp.start()

    def body2(j, carry):
        t0 = 2 * j
        t1 = t0 + 1
        c0 = t0 * NW + w
        c1 = c0 + NW

        @pl.when((t0 >= 2) & (c0 - 2 * NW < NCH))
        def _():
            pltpu.make_async_copy(out.at[pl.ds(0, CH)], rows_a, wsa).wait()

        @pl.when(guard(t0))
        def _():
            for cp in _idx_copies(elem, arom, chg, hct, c0, bufs_a, isa):
                cp.wait()
            for cp in _gathers(We, Wa, Wc, Wh, bufs_a, rows_a, gsa):
                cp.start()

        @pl.when(guard(t1))
        def _():
            for cp in _idx_copies(elem, arom, chg, hct, c1, bufs_b, isb):
                cp.start()

        @pl.when(guard(t0))
        def _():
            for cp in _gathers(We, Wa, Wc, Wh, bufs_a, rows_a, gsa):
                cp.wait()
            pltpu.make_async_copy(rows_a, out.at[pl.ds(c0 * CH, CH)],
                                  wsa).start()

        @pl.when((t1 >= 2) & (c1 - 2 * NW < NCH))
        def _():
            pltpu.make_async_copy(out.at[pl.ds(0, CH)], rows_b, wsb).wait()

        @pl.when(guard(t1))
        def _():
            for cp in _idx_copies(elem, arom, chg, hct, c1, bufs_b, isb):
                cp.wait()
            for cp in _gathers(We, Wa, Wc, Wh, bufs_b, rows_b, gsb):
                cp.start()

        @pl.when(guard(t0 + 2))
        def _():
            for cp in _idx_copies(elem, arom, chg, hct, c0 + 2 * NW,
                                  bufs_a, isa):
                cp.start()

        @pl.when(guard(t1))
        def _():
            for cp in _gathers(We, Wa, Wc, Wh, bufs_b, rows_b, gsb):
                cp.wait()
            pltpu.make_async_copy(rows_b, out.at[pl.ds(c1 * CH, CH)],
                                  wsb).start()

        return carry

    lax.fori_loop(0, (TRIPS + 1) // 2, body2, None)

    @pl.when(w == NW - 1)
    def _():
        base = NCH * CH
        icps = [
            pltpu.make_async_copy(elem.at[pl.ds(base, TAIL)],
                                  iea.at[pl.ds(0, TAIL)], isa),
            pltpu.make_async_copy(arom.at[pl.ds(base, TAIL)],
                                  iaa.at[pl.ds(0, TAIL)], isa),
            pltpu.make_async_copy(chg.at[pl.ds(base, TAIL)],
                                  ica.at[pl.ds(0, TAIL)], isa),
            pltpu.make_async_copy(hct.at[pl.ds(base, TAIL)],
                                  iha.at[pl.ds(0, TAIL)], isa),
        ]
        for cp in icps:
            cp.start()
        for cp in icps:
            cp.wait()
        gcps = [
            pltpu.make_async_copy(We.at[iea.at[pl.ds(0, TAIL)]],
                                  rows_a.at[pl.ds(0, TAIL), pl.ds(0, D)], gsa),
            pltpu.make_async_copy(Wa.at[iaa.at[pl.ds(0, TAIL)]],
                                  rows_a.at[pl.ds(0, TAIL), pl.ds(D, D)], gsa),
            pltpu.make_async_copy(Wc.at[ica.at[pl.ds(0, TAIL)]],
                                  rows_a.at[pl.ds(0, TAIL), pl.ds(2 * D, D)],
                                  gsa),
            pltpu.make_async_copy(Wh.at[iha.at[pl.ds(0, TAIL)]],
                                  rows_a.at[pl.ds(0, TAIL), pl.ds(3 * D, D)],
                                  gsa),
        ]
        for cp in gcps:
            cp.start()
        for cp in gcps:
            cp.wait()
        pltpu.sync_copy(rows_a.at[pl.ds(0, TAIL)], out.at[pl.ds(base, TAIL)])


def kernel(element, aromatic, charge, hcount,
           W_element, W_aromatic, W_charge, W_hcount):
    mesh = plsc.VectorSubcoreMesh(core_axis_name="c", subcore_axis_name="s")
    f = pl.kernel(
        _emb_body,
        mesh=mesh,
        out_type=jax.ShapeDtypeStruct((N, 4 * D), jnp.float32),
        scratch_types=[
            pltpu.VMEM((CH,), jnp.int32),
            pltpu.VMEM((CH,), jnp.int32),
            pltpu.VMEM((CH,), jnp.int32),
            pltpu.VMEM((CH,), jnp.int32),
            pltpu.VMEM((CH,), jnp.int32),
            pltpu.VMEM((CH,), jnp.int32),
            pltpu.VMEM((CH,), jnp.int32),
            pltpu.VMEM((CH,), jnp.int32),
            pltpu.VMEM((CH, 4 * D), jnp.float32),
            pltpu.VMEM((CH, 4 * D), jnp.float32),
            pltpu.SemaphoreType.DMA,
            pltpu.SemaphoreType.DMA,
            pltpu.SemaphoreType.DMA,
            pltpu.SemaphoreType.DMA,
            pltpu.SemaphoreType.DMA,
            pltpu.SemaphoreType.DMA,
        ],
    )
    return f(element, aromatic, charge, hcount,
             W_element, W_aromatic, W_charge, W_hcount)


# speedup vs baseline: 7.5760x; 5.8540x over previous
import jax
import jax.numpy as jnp
from jax import lax
from jax.experimental import pallas as pl
from jax.experimental.pallas import tpu as pltpu
from jax.experimental.pallas import tpu_sc as plsc

N = 100000
D = 128
CH = 96
NG = CH // 16
NCH = N // CH
TAIL = N - NCH * CH
NW = 32
TRIPS = -(-NCH // NW)


def _idx_copies(elem, arom, chg, hct, c, bufs, sem):
    ie, ia, ic, ih = bufs
    return [
        pltpu.make_async_copy(elem.at[pl.ds(c * CH, CH)], ie, sem),
        pltpu.make_async_copy(arom.at[pl.ds(c * CH, CH)], ia, sem),
        pltpu.make_async_copy(chg.at[pl.ds(c * CH, CH)], ic, sem),
        pltpu.make_async_copy(hct.at[pl.ds(c * CH, CH)], ih, sem),
    ]


def _assemble(bufs, rows, ngroups, tabs):
    def grp(g, carry):
        for ibuf, tab, k in tabs:
            v = ibuf[pl.ds(g * 16, 16)]
            for l in range(16):
                s = v[l]
                node = g * 16 + l
                for j in range(8):
                    rows[node, pl.ds(k * D + j * 16, 16)] = (
                        tab[s, pl.ds(j * 16, 16)])
        return carry

    lax.fori_loop(0, ngroups, grp, None)


def _emb_body(elem, arom, chg, hct, We_h, Wa_h, Wc_h, Wh_h, out,
              iea, iaa, ica, iha, ieb, iab, icb, ihb, rows_a, rows_b,
              We, Wa, Wc, Wh,
              isa, isb, wsa, wsb):
    w = lax.axis_index("s") * 2 + lax.axis_index("c")
    bufs_a = (iea, iaa, ica, iha)
    bufs_b = (ieb, iab, icb, ihb)

    pltpu.sync_copy(We_h, We)
    pltpu.sync_copy(Wa_h, Wa)
    pltpu.sync_copy(Wc_h, Wc)
    pltpu.sync_copy(Wh_h, Wh)

    tabs_a = ((iea, We, 0), (iaa, Wa, 1), (ica, Wc, 2), (iha, Wh, 3))
    tabs_b = ((ieb, We, 0), (iab, Wa, 1), (icb, Wc, 2), (ihb, Wh, 3))

    def guard(t):
        return (t < TRIPS) & (t * NW + w < NCH)

    @pl.when(guard(0))
    def _():
        for cp in _idx_copies(elem, arom, chg, hct, 0 * NW + w, bufs_a, isa):
            cp.start()

    @pl.when(guard(1))
    def _():
        for cp in _idx_copies(elem, arom, chg, hct, 1 * NW + w, bufs_b, isb):
            cp.start()

    def half(t, c, bufs, tabs, rows, isem, wsem):
        @pl.when((t >= 2) & (c - 2 * NW < NCH))
        def _():
            pltpu.make_async_copy(out.at[pl.ds(0, CH)], rows, wsem).wait()

        @pl.when(guard(t))
        def _():
            for cp in _idx_copies(elem, arom, chg, hct, c, bufs, isem):
                cp.wait()
            _assemble(bufs, rows, NG, tabs)
            pltpu.make_async_copy(rows, out.at[pl.ds(c * CH, CH)],
                                  wsem).start()

        @pl.when(guard(t + 2))
        def _():
            for cp in _idx_copies(elem, arom, chg, hct, c + 2 * NW,
                                  bufs, isem):
                cp.start()

    def body2(j, carry):
        t0 = 2 * j
        t1 = t0 + 1
        half(t0, t0 * NW + w, bufs_a, tabs_a, rows_a, isa, wsa)
        half(t1, t1 * NW + w, bufs_b, tabs_b, rows_b, isb, wsb)
        return carry

    lax.fori_loop(0, (TRIPS + 1) // 2, body2, None)

    @pl.when(32 * NW + w < NCH)
    def _():
        pltpu.make_async_copy(out.at[pl.ds(0, CH)], rows_a, wsa).wait()

    @pl.when(w == NW - 1)
    def _():
        base = NCH * CH
        icps = [
            pltpu.make_async_copy(elem.at[pl.ds(base, TAIL)],
                                  iea.at[pl.ds(0, TAIL)], isa),
            pltpu.make_async_copy(arom.at[pl.ds(base, TAIL)],
                                  iaa.at[pl.ds(0, TAIL)], isa),
            pltpu.make_async_copy(chg.at[pl.ds(base, TAIL)],
                                  ica.at[pl.ds(0, TAIL)], isa),
            pltpu.make_async_copy(hct.at[pl.ds(base, TAIL)],
                                  iha.at[pl.ds(0, TAIL)], isa),
        ]
        for cp in icps:
            cp.start()
        for cp in icps:
            cp.wait()
        _assemble(bufs_a, rows_a, TAIL // 16, tabs_a)
        pltpu.sync_copy(rows_a.at[pl.ds(0, TAIL)], out.at[pl.ds(base, TAIL)])


def kernel(element, aromatic, charge, hcount,
           W_element, W_aromatic, W_charge, W_hcount):
    mesh = plsc.VectorSubcoreMesh(core_axis_name="c", subcore_axis_name="s")
    f = pl.kernel(
        _emb_body,
        mesh=mesh,
        out_type=jax.ShapeDtypeStruct((N, 4 * D), jnp.float32),
        scratch_types=[
            pltpu.VMEM((CH,), jnp.int32),
            pltpu.VMEM((CH,), jnp.int32),
            pltpu.VMEM((CH,), jnp.int32),
            pltpu.VMEM((CH,), jnp.int32),
            pltpu.VMEM((CH,), jnp.int32),
            pltpu.VMEM((CH,), jnp.int32),
            pltpu.VMEM((CH,), jnp.int32),
            pltpu.VMEM((CH,), jnp.int32),
            pltpu.VMEM((CH, 4 * D), jnp.float32),
            pltpu.VMEM((CH, 4 * D), jnp.float32),
            pltpu.VMEM((100, D), jnp.float32),
            pltpu.VMEM((2, D), jnp.float32),
            pltpu.VMEM((9, D), jnp.float32),
            pltpu.VMEM((9, D), jnp.float32),
            pltpu.SemaphoreType.DMA,
            pltpu.SemaphoreType.DMA,
            pltpu.SemaphoreType.DMA,
            pltpu.SemaphoreType.DMA,
        ],
    )
    return f(element, aromatic, charge, hcount,
             W_element, W_aromatic, W_charge, W_hcount)


# speedup vs baseline: 7.5818x; 1.0008x over previous
import jax
import jax.numpy as jnp
from jax import lax
from jax.experimental import pallas as pl
from jax.experimental.pallas import tpu as pltpu
from jax.experimental.pallas import tpu_sc as plsc

N = 100000
D = 128
CH = 96
NG = CH // 16
NCH = N // CH
TAIL = N - NCH * CH
NW = 32
TRIPS = -(-NCH // NW)


def _idx_copies(elem, arom, chg, hct, c, bufs, sem):
    ie, ia, ic, ih = bufs
    return [
        pltpu.make_async_copy(elem.at[pl.ds(c * CH, CH)], ie, sem),
        pltpu.make_async_copy(arom.at[pl.ds(c * CH, CH)], ia, sem),
        pltpu.make_async_copy(chg.at[pl.ds(c * CH, CH)], ic, sem),
        pltpu.make_async_copy(hct.at[pl.ds(c * CH, CH)], ih, sem),
    ]


def _assemble(bufs, rows, ngroups, tabs):
    def grp(g, carry):
        for ibuf, tab, k in tabs:
            v = ibuf[pl.ds(g * 16, 16)]
            for l in range(16):
                row_src = tab.at[v[l]]
                row_dst = rows.at[g * 16 + l, pl.ds(k * D, D)]
                for j in range(8):
                    row_dst[pl.ds(j * 16, 16)] = row_src[pl.ds(j * 16, 16)]
        return carry

    lax.fori_loop(0, ngroups, grp, None)


def _emb_body(elem, arom, chg, hct, We_h, Wa_h, Wc_h, Wh_h, out,
              iea, iaa, ica, iha, ieb, iab, icb, ihb, rows_a, rows_b,
              We, Wa, Wc, Wh,
              isa, isb, wsa, wsb):
    w = lax.axis_index("s") * 2 + lax.axis_index("c")
    bufs_a = (iea, iaa, ica, iha)
    bufs_b = (ieb, iab, icb, ihb)

    pltpu.sync_copy(We_h, We)
    pltpu.sync_copy(Wa_h, Wa)
    pltpu.sync_copy(Wc_h, Wc)
    pltpu.sync_copy(Wh_h, Wh)

    tabs_a = ((iea, We, 0), (iaa, Wa, 1), (ica, Wc, 2), (iha, Wh, 3))
    tabs_b = ((ieb, We, 0), (iab, Wa, 1), (icb, Wc, 2), (ihb, Wh, 3))

    def guard(t):
        return (t < TRIPS) & (t * NW + w < NCH)

    @pl.when(guard(0))
    def _():
        for cp in _idx_copies(elem, arom, chg, hct, 0 * NW + w, bufs_a, isa):
            cp.start()

    @pl.when(guard(1))
    def _():
        for cp in _idx_copies(elem, arom, chg, hct, 1 * NW + w, bufs_b, isb):
            cp.start()

    def half(t, c, bufs, tabs, rows, isem, wsem):
        @pl.when((t >= 2) & (c - 2 * NW < NCH))
        def _():
            pltpu.make_async_copy(out.at[pl.ds(0, CH)], rows, wsem).wait()

        @pl.when(guard(t))
        def _():
            for cp in _idx_copies(elem, arom, chg, hct, c, bufs, isem):
                cp.wait()
            _assemble(bufs, rows, NG, tabs)
            pltpu.make_async_copy(rows, out.at[pl.ds(c * CH, CH)],
                                  wsem).start()

        @pl.when(guard(t + 2))
        def _():
            for cp in _idx_copies(elem, arom, chg, hct, c + 2 * NW,
                                  bufs, isem):
                cp.start()

    def body2(j, carry):
        t0 = 2 * j
        t1 = t0 + 1
        half(t0, t0 * NW + w, bufs_a, tabs_a, rows_a, isa, wsa)
        half(t1, t1 * NW + w, bufs_b, tabs_b, rows_b, isb, wsb)
        return carry

    lax.fori_loop(0, (TRIPS + 1) // 2, body2, None)

    @pl.when(32 * NW + w < NCH)
    def _():
        pltpu.make_async_copy(out.at[pl.ds(0, CH)], rows_a, wsa).wait()

    @pl.when(w == NW - 1)
    def _():
        base = NCH * CH
        icps = [
            pltpu.make_async_copy(elem.at[pl.ds(base, TAIL)],
                                  iea.at[pl.ds(0, TAIL)], isa),
            pltpu.make_async_copy(arom.at[pl.ds(base, TAIL)],
                                  iaa.at[pl.ds(0, TAIL)], isa),
            pltpu.make_async_copy(chg.at[pl.ds(base, TAIL)],
                                  ica.at[pl.ds(0, TAIL)], isa),
            pltpu.make_async_copy(hct.at[pl.ds(base, TAIL)],
                                  iha.at[pl.ds(0, TAIL)], isa),
        ]
        for cp in icps:
            cp.start()
        for cp in icps:
            cp.wait()
        _assemble(bufs_a, rows_a, TAIL // 16, tabs_a)
        pltpu.sync_copy(rows_a.at[pl.ds(0, TAIL)], out.at[pl.ds(base, TAIL)])


def kernel(element, aromatic, charge, hcount,
           W_element, W_aromatic, W_charge, W_hcount):
    mesh = plsc.VectorSubcoreMesh(core_axis_name="c", subcore_axis_name="s")
    f = pl.kernel(
        _emb_body,
        mesh=mesh,
        out_type=jax.ShapeDtypeStruct((N, 4 * D), jnp.float32),
        scratch_types=[
            pltpu.VMEM((CH,), jnp.int32),
            pltpu.VMEM((CH,), jnp.int32),
            pltpu.VMEM((CH,), jnp.int32),
            pltpu.VMEM((CH,), jnp.int32),
            pltpu.VMEM((CH,), jnp.int32),
            pltpu.VMEM((CH,), jnp.int32),
            pltpu.VMEM((CH,), jnp.int32),
            pltpu.VMEM((CH,), jnp.int32),
            pltpu.VMEM((CH, 4 * D), jnp.float32),
            pltpu.VMEM((CH, 4 * D), jnp.float32),
            pltpu.VMEM((100, D), jnp.float32),
            pltpu.VMEM((2, D), jnp.float32),
            pltpu.VMEM((9, D), jnp.float32),
            pltpu.VMEM((9, D), jnp.float32),
            pltpu.SemaphoreType.DMA,
            pltpu.SemaphoreType.DMA,
            pltpu.SemaphoreType.DMA,
            pltpu.SemaphoreType.DMA,
        ],
    )
    return f(element, aromatic, charge, hcount,
             W_element, W_aromatic, W_charge, W_hcount)
